# baseline (speedup 1.0000x reference)
import functools

import jax
import jax.numpy as jnp
from jax import lax
from jax.experimental import pallas as pl
from jax.experimental.pallas import tpu as pltpu
from jax.experimental.pallas import tpu_sc as plsc

QBLK = 512
Q_SC = 1024
M = 1024
N = 4096
B = 16
NWORK = 32
CH = B * Q_SC // NWORK


def _threenn_block(u_ref, kt_ref, dist_ref, idx_ref):
    u = u_ref[0]
    kt = kt_ref[0]
    ux, uy, uz = u[:, 0:1], u[:, 1:2], u[:, 2:3]
    kx, ky, kz = kt[0:1, :], kt[1:2, :], kt[2:3, :]

    dotm2 = jnp.dot((-2.0 * u).astype(jnp.bfloat16), kt.astype(jnp.bfloat16),
                    preferred_element_type=jnp.float32)
    su = ux * ux + uy * uy + uz * uz
    sk = kx * kx + ky * ky + kz * kz
    d = dotm2 + su + sk

    iota = jax.lax.broadcasted_iota(jnp.int32, d.shape, 1).astype(jnp.float32)
    big = jnp.float32(d.shape[-1])
    for k in range(3):
        mn = jnp.min(d, axis=1, keepdims=True)
        im = jnp.min(jnp.where(d == mn, iota, big), axis=1, keepdims=True)
        dist_ref[0, :, pl.ds(k, 1)] = mn
        idx_ref[0, :, pl.ds(k, 1)] = im.astype(jnp.int32)
        if k < 2:
            d = jnp.where(iota == im, jnp.inf, d)


def _tc_threenn(unknown, known):
    b, n, _ = unknown.shape
    m = known.shape[1]
    kt = known.transpose(0, 2, 1)
    grid = (b, n // QBLK)
    return pl.pallas_call(
        _threenn_block,
        grid=grid,
        in_specs=[
            pl.BlockSpec((1, QBLK, 3), lambda i, j: (i, j, 0)),
            pl.BlockSpec((1, 3, m), lambda i, j: (i, 0, 0)),
        ],
        out_specs=[
            pl.BlockSpec((1, QBLK, 3), lambda i, j: (i, j, 0)),
            pl.BlockSpec((1, QBLK, 3), lambda i, j: (i, j, 0)),
        ],
        out_shape=[
            jax.ShapeDtypeStruct((b, n, 3), jnp.float32),
            jax.ShapeDtypeStruct((b, n, 3), jnp.int32),
        ],
    )(unknown, kt)


def _sc_body(qx_h, qy_h, qz_h, su_h, kx_h, ky_h, kz_h, sk_h,
             dist_h, idx_h,
             qx_v, qy_v, qz_v, su_v, kx_v, ky_v, kz_v, sk_v,
             dist_v, idx_v):
    wid = lax.axis_index("s") * 2 + lax.axis_index("c")
    qbase = wid * CH
    kbase = (qbase // Q_SC) * (M * 16)

    pltpu.sync_copy(qx_h.at[pl.ds(qbase, CH)], qx_v)
    pltpu.sync_copy(qy_h.at[pl.ds(qbase, CH)], qy_v)
    pltpu.sync_copy(qz_h.at[pl.ds(qbase, CH)], qz_v)
    pltpu.sync_copy(su_h.at[pl.ds(qbase, CH)], su_v)
    pltpu.sync_copy(kx_h.at[pl.ds(kbase, M * 16)], kx_v)
    pltpu.sync_copy(ky_h.at[pl.ds(kbase, M * 16)], ky_v)
    pltpu.sync_copy(kz_h.at[pl.ds(kbase, M * 16)], kz_v)
    pltpu.sync_copy(sk_h.at[pl.ds(kbase, M * 16)], sk_v)

    inf16 = jnp.full((16,), jnp.inf, jnp.float32)
    zero16 = jnp.zeros((16,), jnp.int32)
    mone16 = jnp.full((16,), -1, jnp.int32)
    one16 = jnp.full((16,), 1, jnp.int32)

    UNROLL = 16

    def per_group(g, _):
        goff = g * 16
        qxv = qx_v[pl.ds(goff, 16)]
        qyv = qy_v[pl.ds(goff, 16)]
        qzv = qz_v[pl.ds(goff, 16)]
        suv = su_v[pl.ds(goff, 16)]

        def per_cand(j, carry):
            m1, m2, m3, i1, i2, i3, iv = carry
            for jl in range(UNROLL):
                off = (j * UNROLL + jl) * 16
                kxs = kx_v[pl.ds(off, 16)]
                kys = ky_v[pl.ds(off, 16)]
                kzs = kz_v[pl.ds(off, 16)]
                sks = sk_v[pl.ds(off, 16)]
                d = qxv * kxs + qyv * kys + qzv * kzs + suv + sks
                iv = iv + one16
                c1 = d < m1
                c2 = d < m2
                c3 = d < m3
                m3n = jnp.where(c3, jnp.where(c2, m2, d), m3)
                i3n = jnp.where(c3, jnp.where(c2, i2, iv), i3)
                m2 = jnp.where(c2, jnp.where(c1, m1, d), m2)
                i2 = jnp.where(c2, jnp.where(c1, i1, iv), i2)
                m1 = jnp.where(c1, d, m1)
                i1 = jnp.where(c1, iv, i1)
                m3, i3 = m3n, i3n
            return m1, m2, m3, i1, i2, i3, iv

        m1, m2, m3, i1, i2, i3, _ = lax.fori_loop(
            0, M // UNROLL, per_cand,
            (inf16, inf16, inf16, zero16, zero16, zero16, mone16))

        dist_v[pl.ds(goff, 16)] = m1
        dist_v[pl.ds(CH + goff, 16)] = m2
        dist_v[pl.ds(2 * CH + goff, 16)] = m3
        idx_v[pl.ds(goff, 16)] = i1
        idx_v[pl.ds(CH + goff, 16)] = i2
        idx_v[pl.ds(2 * CH + goff, 16)] = i3
        return 0

    lax.fori_loop(0, CH // 16, per_group, 0)

    pltpu.sync_copy(dist_v, dist_h.at[pl.ds(qbase * 3, CH * 3)])
    pltpu.sync_copy(idx_v, idx_h.at[pl.ds(qbase * 3, CH * 3)])


def _sc_threenn(unknown, known):
    bf = lambda x: lax.optimization_barrier(
        x.astype(jnp.bfloat16)).astype(jnp.float32)
    um2 = bf(-2.0 * unknown)
    kb = bf(known)
    qx = um2[..., 0].reshape(-1)
    qy = um2[..., 1].reshape(-1)
    qz = um2[..., 2].reshape(-1)
    su = (unknown[..., 0] * unknown[..., 0]
          + unknown[..., 1] * unknown[..., 1]
          + unknown[..., 2] * unknown[..., 2]).reshape(-1)
    sk = (known[..., 0] * known[..., 0]
          + known[..., 1] * known[..., 1]
          + known[..., 2] * known[..., 2])
    splat = lambda a: jnp.broadcast_to(a[..., None], a.shape + (16,)).reshape(-1)
    kxs = splat(kb[..., 0])
    kys = splat(kb[..., 1])
    kzs = splat(kb[..., 2])
    sks = splat(sk)

    mesh = plsc.VectorSubcoreMesh(core_axis_name="c", subcore_axis_name="s")
    fn = pl.kernel(
        _sc_body,
        mesh=mesh,
        out_type=[
            jax.ShapeDtypeStruct((B * Q_SC * 3,), jnp.float32),
            jax.ShapeDtypeStruct((B * Q_SC * 3,), jnp.int32),
        ],
        scratch_types=[
            pltpu.VMEM((CH,), jnp.float32),
            pltpu.VMEM((CH,), jnp.float32),
            pltpu.VMEM((CH,), jnp.float32),
            pltpu.VMEM((CH,), jnp.float32),
            pltpu.VMEM((M * 16,), jnp.float32),
            pltpu.VMEM((M * 16,), jnp.float32),
            pltpu.VMEM((M * 16,), jnp.float32),
            pltpu.VMEM((M * 16,), jnp.float32),
            pltpu.VMEM((CH * 3,), jnp.float32),
            pltpu.VMEM((CH * 3,), jnp.int32),
        ],
    )
    dist, idx = fn(qx, qy, qz, su, kxs, kys, kzs, sks)
    wpb = Q_SC // CH
    def unpack(a):
        return (a.reshape(B, wpb, 3, CH)
                 .transpose(0, 1, 3, 2)
                 .reshape(B, Q_SC, 3))
    return unpack(dist), unpack(idx)


@jax.jit
def kernel(unknown, known):
    n_tc = N - Q_SC
    dist_tc, idx_tc = _tc_threenn(unknown[:, :n_tc], known)
    dist_sc, idx_sc = _sc_threenn(unknown[:, n_tc:], known)
    dist = jnp.concatenate([dist_tc, dist_sc], axis=1)
    idx = jnp.concatenate([idx_tc, idx_sc], axis=1)
    return dist, idx


# speedup vs baseline: 1.0343x; 1.0343x over previous
import functools

import jax
import jax.numpy as jnp
from jax import lax
from jax.experimental import pallas as pl
from jax.experimental.pallas import tpu as pltpu
from jax.experimental.pallas import tpu_sc as plsc

QBLK = 1024
Q_SC = 1024
M = 1024
N = 4096
B = 16
NWORK = 32
CH = B * Q_SC // NWORK


def _threenn_block(u_ref, kt_ref, dist_ref, idx_ref):
    u = u_ref[0]
    kt = kt_ref[0]
    ux, uy, uz = u[:, 0:1], u[:, 1:2], u[:, 2:3]
    kx, ky, kz = kt[0:1, :], kt[1:2, :], kt[2:3, :]

    dotm2 = jnp.dot((-2.0 * u).astype(jnp.bfloat16), kt.astype(jnp.bfloat16),
                    preferred_element_type=jnp.float32)
    su = ux * ux + uy * uy + uz * uz
    sk = kx * kx + ky * ky + kz * kz
    d = dotm2 + su + sk

    iota = jax.lax.broadcasted_iota(jnp.int32, d.shape, 1).astype(jnp.float32)
    big = jnp.float32(d.shape[-1])
    for k in range(3):
        mn = jnp.min(d, axis=1, keepdims=True)
        im = jnp.min(jnp.where(d == mn, iota, big), axis=1, keepdims=True)
        dist_ref[0, :, pl.ds(k, 1)] = mn
        idx_ref[0, :, pl.ds(k, 1)] = im.astype(jnp.int32)
        if k < 2:
            d = jnp.where(iota == im, jnp.inf, d)


def _tc_threenn(unknown, known):
    b, n, _ = unknown.shape
    m = known.shape[1]
    kt = known.transpose(0, 2, 1)
    grid = (b, n // QBLK)
    return pl.pallas_call(
        _threenn_block,
        grid=grid,
        in_specs=[
            pl.BlockSpec((1, QBLK, 3), lambda i, j: (i, j, 0)),
            pl.BlockSpec((1, 3, m), lambda i, j: (i, 0, 0)),
        ],
        out_specs=[
            pl.BlockSpec((1, QBLK, 3), lambda i, j: (i, j, 0)),
            pl.BlockSpec((1, QBLK, 3), lambda i, j: (i, j, 0)),
        ],
        out_shape=[
            jax.ShapeDtypeStruct((b, n, 3), jnp.float32),
            jax.ShapeDtypeStruct((b, n, 3), jnp.int32),
        ],
    )(unknown, kt)


def _sc_body(qx_h, qy_h, qz_h, su_h, kx_h, ky_h, kz_h, sk_h,
             dist_h, idx_h,
             qx_v, qy_v, qz_v, su_v, kx_v, ky_v, kz_v, sk_v,
             dist_v, idx_v):
    wid = lax.axis_index("s") * 2 + lax.axis_index("c")
    qbase = wid * CH
    kbase = (qbase // Q_SC) * (M * 16)

    pltpu.sync_copy(qx_h.at[pl.ds(qbase, CH)], qx_v)
    pltpu.sync_copy(qy_h.at[pl.ds(qbase, CH)], qy_v)
    pltpu.sync_copy(qz_h.at[pl.ds(qbase, CH)], qz_v)
    pltpu.sync_copy(su_h.at[pl.ds(qbase, CH)], su_v)
    pltpu.sync_copy(kx_h.at[pl.ds(kbase, M * 16)], kx_v)
    pltpu.sync_copy(ky_h.at[pl.ds(kbase, M * 16)], ky_v)
    pltpu.sync_copy(kz_h.at[pl.ds(kbase, M * 16)], kz_v)
    pltpu.sync_copy(sk_h.at[pl.ds(kbase, M * 16)], sk_v)

    inf16 = jnp.full((16,), jnp.inf, jnp.float32)
    zero16 = jnp.zeros((16,), jnp.int32)
    four16 = jnp.full((16,), 4, jnp.int32)

    UNROLL = 16
    NCHAIN = 4

    def lt(va, ia, vb, ib):
        return (va < vb) | ((va == vb) & (ia < ib))

    def merge3(a, b):
        a1v, a2v, a3v, a1i, a2i, a3i = a
        b1v, b2v, b3v, b1i, b2i, b3i = b
        w1 = lt(a1v, a1i, b1v, b1i)
        o1v = jnp.where(w1, a1v, b1v)
        o1i = jnp.where(w1, a1i, b1i)
        ahv = jnp.where(w1, a2v, a1v)
        ahi = jnp.where(w1, a2i, a1i)
        anv = jnp.where(w1, a3v, a2v)
        ani = jnp.where(w1, a3i, a2i)
        bhv = jnp.where(w1, b1v, b2v)
        bhi = jnp.where(w1, b1i, b2i)
        bnv = jnp.where(w1, b2v, b3v)
        bni = jnp.where(w1, b2i, b3i)
        w2 = lt(ahv, ahi, bhv, bhi)
        o2v = jnp.where(w2, ahv, bhv)
        o2i = jnp.where(w2, ahi, bhi)
        ah2v = jnp.where(w2, anv, ahv)
        ah2i = jnp.where(w2, ani, ahi)
        bh2v = jnp.where(w2, bhv, bnv)
        bh2i = jnp.where(w2, bhi, bni)
        w3 = lt(ah2v, ah2i, bh2v, bh2i)
        o3v = jnp.where(w3, ah2v, bh2v)
        o3i = jnp.where(w3, ah2i, bh2i)
        return o1v, o2v, o3v, o1i, o2i, o3i

    def per_group(g, _):
        goff = g * 16
        qxv = qx_v[pl.ds(goff, 16)]
        qyv = qy_v[pl.ds(goff, 16)]
        qzv = qz_v[pl.ds(goff, 16)]
        suv = su_v[pl.ds(goff, 16)]

        def per_cand(j, carry):
            chains = list(carry)
            for jl in range(UNROLL):
                c = jl % NCHAIN
                m1, m2, m3, i1, i2, i3, iv = chains[c]
                off = (j * UNROLL + jl) * 16
                kxs = kx_v[pl.ds(off, 16)]
                kys = ky_v[pl.ds(off, 16)]
                kzs = kz_v[pl.ds(off, 16)]
                sks = sk_v[pl.ds(off, 16)]
                d = qxv * kxs + qyv * kys + qzv * kzs + suv + sks
                iv = iv + four16
                c1 = d < m1
                c2 = d < m2
                c3 = d < m3
                m3n = jnp.where(c3, jnp.where(c2, m2, d), m3)
                i3n = jnp.where(c3, jnp.where(c2, i2, iv), i3)
                m2 = jnp.where(c2, jnp.where(c1, m1, d), m2)
                i2 = jnp.where(c2, jnp.where(c1, i1, iv), i2)
                m1 = jnp.where(c1, d, m1)
                i1 = jnp.where(c1, iv, i1)
                chains[c] = (m1, m2, m3n, i1, i2, i3n, iv)
            return tuple(chains)

        init = tuple(
            (inf16, inf16, inf16, zero16, zero16, zero16,
             jnp.full((16,), c - NCHAIN, jnp.int32))
            for c in range(NCHAIN))
        chains = lax.fori_loop(0, M // UNROLL, per_cand, init)

        tri = [ch[:6] for ch in chains]
        ab = merge3(tri[0], tri[1])
        cd = merge3(tri[2], tri[3])
        m1, m2, m3, i1, i2, i3 = merge3(ab, cd)

        dist_v[pl.ds(goff, 16)] = m1
        dist_v[pl.ds(CH + goff, 16)] = m2
        dist_v[pl.ds(2 * CH + goff, 16)] = m3
        idx_v[pl.ds(goff, 16)] = i1
        idx_v[pl.ds(CH + goff, 16)] = i2
        idx_v[pl.ds(2 * CH + goff, 16)] = i3
        return 0

    lax.fori_loop(0, CH // 16, per_group, 0)

    pltpu.sync_copy(dist_v, dist_h.at[pl.ds(qbase * 3, CH * 3)])
    pltpu.sync_copy(idx_v, idx_h.at[pl.ds(qbase * 3, CH * 3)])


def _sc_threenn(unknown, known):
    bf = lambda x: lax.optimization_barrier(
        x.astype(jnp.bfloat16)).astype(jnp.float32)
    um2 = bf(-2.0 * unknown)
    kb = bf(known)
    qx = um2[..., 0].reshape(-1)
    qy = um2[..., 1].reshape(-1)
    qz = um2[..., 2].reshape(-1)
    su = (unknown[..., 0] * unknown[..., 0]
          + unknown[..., 1] * unknown[..., 1]
          + unknown[..., 2] * unknown[..., 2]).reshape(-1)
    sk = (known[..., 0] * known[..., 0]
          + known[..., 1] * known[..., 1]
          + known[..., 2] * known[..., 2])
    splat = lambda a: jnp.broadcast_to(a[..., None], a.shape + (16,)).reshape(-1)
    kxs = splat(kb[..., 0])
    kys = splat(kb[..., 1])
    kzs = splat(kb[..., 2])
    sks = splat(sk)

    mesh = plsc.VectorSubcoreMesh(core_axis_name="c", subcore_axis_name="s")
    fn = pl.kernel(
        _sc_body,
        mesh=mesh,
        out_type=[
            jax.ShapeDtypeStruct((B * Q_SC * 3,), jnp.float32),
            jax.ShapeDtypeStruct((B * Q_SC * 3,), jnp.int32),
        ],
        scratch_types=[
            pltpu.VMEM((CH,), jnp.float32),
            pltpu.VMEM((CH,), jnp.float32),
            pltpu.VMEM((CH,), jnp.float32),
            pltpu.VMEM((CH,), jnp.float32),
            pltpu.VMEM((M * 16,), jnp.float32),
            pltpu.VMEM((M * 16,), jnp.float32),
            pltpu.VMEM((M * 16,), jnp.float32),
            pltpu.VMEM((M * 16,), jnp.float32),
            pltpu.VMEM((CH * 3,), jnp.float32),
            pltpu.VMEM((CH * 3,), jnp.int32),
        ],
    )
    dist, idx = fn(qx, qy, qz, su, kxs, kys, kzs, sks)
    wpb = Q_SC // CH
    def unpack(a):
        return (a.reshape(B, wpb, 3, CH)
                 .transpose(0, 1, 3, 2)
                 .reshape(B, Q_SC, 3))
    return unpack(dist), unpack(idx)


@jax.jit
def kernel(unknown, known):
    n_tc = N - Q_SC
    dist_tc, idx_tc = _tc_threenn(unknown[:, :n_tc], known)
    dist_sc, idx_sc = _sc_threenn(unknown[:, n_tc:], known)
    dist = jnp.concatenate([dist_tc, dist_sc], axis=1)
    idx = jnp.concatenate([idx_tc, idx_sc], axis=1)
    return dist, idx


# speedup vs baseline: 1.2182x; 1.1777x over previous
import jax
import jax.numpy as jnp
from jax import lax
from jax.experimental import pallas as pl
from jax.experimental.pallas import tpu as pltpu
from jax.experimental.pallas import tpu_sc as plsc

QBLK = 1024
NB_SC = 4
M = 1024
N = 4096
B = 16
NWORK = 32
CH = NB_SC * N // NWORK


def _threenn_block(u_ref, kt_ref, dist_ref, idx_ref):
    u = u_ref[0]
    kt = kt_ref[0]
    ux, uy, uz = u[:, 0:1], u[:, 1:2], u[:, 2:3]
    kx, ky, kz = kt[0:1, :], kt[1:2, :], kt[2:3, :]

    dotm2 = jnp.dot((-2.0 * u).astype(jnp.bfloat16), kt.astype(jnp.bfloat16),
                    preferred_element_type=jnp.float32)
    su = ux * ux + uy * uy + uz * uz
    sk = kx * kx + ky * ky + kz * kz
    d = dotm2 + su + sk

    iota = jax.lax.broadcasted_iota(jnp.int32, d.shape, 1).astype(jnp.float32)
    big = jnp.float32(d.shape[-1])
    for k in range(3):
        mn = jnp.min(d, axis=1, keepdims=True)
        im = jnp.min(jnp.where(d == mn, iota, big), axis=1, keepdims=True)
        dist_ref[0, :, pl.ds(k, 1)] = mn
        idx_ref[0, :, pl.ds(k, 1)] = im.astype(jnp.int32)
        if k < 2:
            d = jnp.where(iota == im, jnp.inf, d)


def _tc_threenn(unknown, known):
    b, n, _ = unknown.shape
    m = known.shape[1]
    kt = known.transpose(0, 2, 1)
    grid = (b, n // QBLK)
    return pl.pallas_call(
        _threenn_block,
        grid=grid,
        in_specs=[
            pl.BlockSpec((1, QBLK, 3), lambda i, j: (i, j, 0)),
            pl.BlockSpec((1, 3, m), lambda i, j: (i, 0, 0)),
        ],
        out_specs=[
            pl.BlockSpec((1, QBLK, 3), lambda i, j: (i, j, 0)),
            pl.BlockSpec((1, QBLK, 3), lambda i, j: (i, j, 0)),
        ],
        out_shape=[
            jax.ShapeDtypeStruct((b, n, 3), jnp.float32),
            jax.ShapeDtypeStruct((b, n, 3), jnp.int32),
        ],
    )(unknown, kt)


def _sc_body(qx_h, qy_h, qz_h, su_h, kx_h, ky_h, kz_h, sk_h,
             dist_h, idx_h,
             qx_v, qy_v, qz_v, su_v, kx_v, ky_v, kz_v, sk_v,
             dist_v, idx_v):
    wid = lax.axis_index("s") * 2 + lax.axis_index("c")
    qbase = wid * CH
    kbase = (qbase // N) * M

    pltpu.sync_copy(qx_h.at[pl.ds(qbase, CH)], qx_v)
    pltpu.sync_copy(qy_h.at[pl.ds(qbase, CH)], qy_v)
    pltpu.sync_copy(qz_h.at[pl.ds(qbase, CH)], qz_v)
    pltpu.sync_copy(su_h.at[pl.ds(qbase, CH)], su_v)
    pltpu.sync_copy(kx_h.at[pl.ds(kbase, M)], kx_v)
    pltpu.sync_copy(ky_h.at[pl.ds(kbase, M)], ky_v)
    pltpu.sync_copy(kz_h.at[pl.ds(kbase, M)], kz_v)
    pltpu.sync_copy(sk_h.at[pl.ds(kbase, M)], sk_v)

    inf16 = jnp.full((16,), jnp.inf, jnp.float32)
    zero16 = jnp.zeros((16,), jnp.int32)
    four16 = jnp.full((16,), 4, jnp.int32)

    UNROLL = 16
    NCHAIN = 4

    def lt(va, ia, vb, ib):
        return (va < vb) | ((va == vb) & (ia < ib))

    def merge3(a, b):
        a1v, a2v, a3v, a1i, a2i, a3i = a
        b1v, b2v, b3v, b1i, b2i, b3i = b
        w1 = lt(a1v, a1i, b1v, b1i)
        o1v = jnp.where(w1, a1v, b1v)
        o1i = jnp.where(w1, a1i, b1i)
        ahv = jnp.where(w1, a2v, a1v)
        ahi = jnp.where(w1, a2i, a1i)
        anv = jnp.where(w1, a3v, a2v)
        ani = jnp.where(w1, a3i, a2i)
        bhv = jnp.where(w1, b1v, b2v)
        bhi = jnp.where(w1, b1i, b2i)
        bnv = jnp.where(w1, b2v, b3v)
        bni = jnp.where(w1, b2i, b3i)
        w2 = lt(ahv, ahi, bhv, bhi)
        o2v = jnp.where(w2, ahv, bhv)
        o2i = jnp.where(w2, ahi, bhi)
        ah2v = jnp.where(w2, anv, ahv)
        ah2i = jnp.where(w2, ani, ahi)
        bh2v = jnp.where(w2, bhv, bnv)
        bh2i = jnp.where(w2, bhi, bni)
        w3 = lt(ah2v, ah2i, bh2v, bh2i)
        o3v = jnp.where(w3, ah2v, bh2v)
        o3i = jnp.where(w3, ah2i, bh2i)
        return o1v, o2v, o3v, o1i, o2i, o3i

    def per_group(g, _):
        goff = g * 16
        qxv = qx_v[pl.ds(goff, 16)]
        qyv = qy_v[pl.ds(goff, 16)]
        qzv = qz_v[pl.ds(goff, 16)]
        suv = su_v[pl.ds(goff, 16)]

        def per_cand(j, carry):
            chains = list(carry)
            goff16 = j * UNROLL
            kxg = kx_v[pl.ds(goff16, UNROLL)]
            kyg = ky_v[pl.ds(goff16, UNROLL)]
            kzg = kz_v[pl.ds(goff16, UNROLL)]
            skg = sk_v[pl.ds(goff16, UNROLL)]
            for jl in range(UNROLL):
                c = jl % NCHAIN
                m1, m2, m3, i1, i2, i3, iv = chains[c]
                kxs = jnp.full((16,), kxg[jl], jnp.float32)
                kys = jnp.full((16,), kyg[jl], jnp.float32)
                kzs = jnp.full((16,), kzg[jl], jnp.float32)
                sks = jnp.full((16,), skg[jl], jnp.float32)
                d = qxv * kxs + qyv * kys + qzv * kzs + suv + sks
                iv = iv + four16
                c1 = d < m1
                c2 = d < m2
                c3 = d < m3
                m3n = jnp.where(c3, jnp.where(c2, m2, d), m3)
                i3n = jnp.where(c3, jnp.where(c2, i2, iv), i3)
                m2 = jnp.where(c2, jnp.where(c1, m1, d), m2)
                i2 = jnp.where(c2, jnp.where(c1, i1, iv), i2)
                m1 = jnp.where(c1, d, m1)
                i1 = jnp.where(c1, iv, i1)
                chains[c] = (m1, m2, m3n, i1, i2, i3n, iv)
            return tuple(chains)

        init = tuple(
            (inf16, inf16, inf16, zero16, zero16, zero16,
             jnp.full((16,), c - NCHAIN, jnp.int32))
            for c in range(NCHAIN))
        chains = lax.fori_loop(0, M // UNROLL, per_cand, init)

        tri = [ch[:6] for ch in chains]
        ab = merge3(tri[0], tri[1])
        cd = merge3(tri[2], tri[3])
        m1, m2, m3, i1, i2, i3 = merge3(ab, cd)

        dist_v[pl.ds(goff, 16)] = m1
        dist_v[pl.ds(CH + goff, 16)] = m2
        dist_v[pl.ds(2 * CH + goff, 16)] = m3
        idx_v[pl.ds(goff, 16)] = i1
        idx_v[pl.ds(CH + goff, 16)] = i2
        idx_v[pl.ds(2 * CH + goff, 16)] = i3
        return 0

    lax.fori_loop(0, CH // 16, per_group, 0)

    pltpu.sync_copy(dist_v, dist_h.at[pl.ds(qbase * 3, CH * 3)])
    pltpu.sync_copy(idx_v, idx_h.at[pl.ds(qbase * 3, CH * 3)])


def _sc_threenn(unknown, known):
    bf = lambda x: lax.optimization_barrier(
        x.astype(jnp.bfloat16)).astype(jnp.float32)
    um2 = bf(-2.0 * unknown)
    kb = bf(known)
    qx = um2[..., 0].reshape(-1)
    qy = um2[..., 1].reshape(-1)
    qz = um2[..., 2].reshape(-1)
    su = (unknown[..., 0] * unknown[..., 0]
          + unknown[..., 1] * unknown[..., 1]
          + unknown[..., 2] * unknown[..., 2]).reshape(-1)
    sk = (known[..., 0] * known[..., 0]
          + known[..., 1] * known[..., 1]
          + known[..., 2] * known[..., 2]).reshape(-1)
    kxs = kb[..., 0].reshape(-1)
    kys = kb[..., 1].reshape(-1)
    kzs = kb[..., 2].reshape(-1)

    mesh = plsc.VectorSubcoreMesh(core_axis_name="c", subcore_axis_name="s")
    fn = pl.kernel(
        _sc_body,
        mesh=mesh,
        out_type=[
            jax.ShapeDtypeStruct((NB_SC * N * 3,), jnp.float32),
            jax.ShapeDtypeStruct((NB_SC * N * 3,), jnp.int32),
        ],
        scratch_types=[
            pltpu.VMEM((CH,), jnp.float32),
            pltpu.VMEM((CH,), jnp.float32),
            pltpu.VMEM((CH,), jnp.float32),
            pltpu.VMEM((CH,), jnp.float32),
            pltpu.VMEM((M,), jnp.float32),
            pltpu.VMEM((M,), jnp.float32),
            pltpu.VMEM((M,), jnp.float32),
            pltpu.VMEM((M,), jnp.float32),
            pltpu.VMEM((CH * 3,), jnp.float32),
            pltpu.VMEM((CH * 3,), jnp.int32),
        ],
    )
    dist, idx = fn(qx, qy, qz, su, kxs, kys, kzs, sk)
    wpb = N // CH
    def unpack(a):
        return (a.reshape(NB_SC, wpb, 3, CH)
                 .transpose(0, 1, 3, 2)
                 .reshape(NB_SC, N, 3))
    return unpack(dist), unpack(idx)


@jax.jit
def kernel(unknown, known):
    nb_tc = B - NB_SC
    dist_tc, idx_tc = _tc_threenn(unknown[:nb_tc], known[:nb_tc])
    dist_sc, idx_sc = _sc_threenn(unknown[nb_tc:], known[nb_tc:])
    dist = jnp.concatenate([dist_tc, dist_sc], axis=0)
    idx = jnp.concatenate([idx_tc, idx_sc], axis=0)
    return dist, idx
